# baseline (speedup 1.0000x reference)
import jax
import jax.numpy as jnp
from jax.experimental import pallas as pl
from jax.experimental.pallas import tpu as pltpu

_ROWS = 8192
_COLS = 1024


def _copy_body(x_ref, o_ref):
    o_ref[...] = x_ref[...]


def kernel(x, embed_weight):
    del embed_weight
    flat = x.reshape(_ROWS, _COLS)
    out = pl.pallas_call(
        _copy_body,
        out_shape=jax.ShapeDtypeStruct(flat.shape, flat.dtype),
        grid=(4,),
        in_specs=[pl.BlockSpec((2048, _COLS), lambda i: (i, 0))],
        out_specs=pl.BlockSpec((2048, _COLS), lambda i: (i, 0)),
        compiler_params=pltpu.CompilerParams(
            dimension_semantics=("parallel",),
        ),
    )(flat)
    return out.reshape(x.shape)
